# baseline (speedup 1.0000x reference)
import jax
import jax.numpy as jnp
from jax import lax
from jax.experimental import pallas as pl
from jax.experimental.pallas import tpu as pltpu
from jax.experimental.pallas import tpu_sc as plsc

N = 16384
M = 32
L = 16
NV = N // L
NH = NV // 2
R1_BITS = 12
R1 = 1 << R1_BITS
R23_BITS = 9
R23 = 1 << R23_BITS
IDX_BITS = 14
IDX_MASK = (1 << IDX_BITS) - 1

_LN2 = 0.6931471805599453
_SQRT2 = 1.4142135623730951


def _log_poly(x):
  bits = plsc.bitcast(x, jnp.int32)
  e = jnp.right_shift(bits, 23) - 127
  m = plsc.bitcast(
      jnp.bitwise_or(jnp.bitwise_and(bits, 0x7FFFFF), 0x3F800000),
      jnp.float32)
  big = m > _SQRT2
  m = jnp.where(big, m * 0.5, m)
  e = e + jnp.where(big, 1, 0)
  s = (m - 1.0) / (m + 1.0)
  s2 = s * s
  p = 1.0 + s2 * (1.0 / 3.0 + s2 * (0.2 + s2 * (1.0 / 7.0 + s2 / 9.0)))
  return e.astype(jnp.float32) * _LN2 + 2.0 * s * p


def _sc_body(time_hbm, risk_hbm, ev_hbm, out_hbm,
             time_c, risk_c, ev_c, key_a, work_b,
             h1a, h1b, h2a, h2b, h3a, h3b, offa, offb, pvec):
  wid = lax.axis_index("s") * 2 + lax.axis_index("c")

  pltpu.sync_copy(time_hbm.at[wid], time_c)
  pltpu.sync_copy(risk_hbm.at[wid], risk_c)
  pltpu.sync_copy(ev_hbm.at[wid], ev_c)

  zero_i = jnp.zeros((L,), jnp.int32)
  zero_f = jnp.zeros((L,), jnp.float32)
  one_i = jnp.ones((L,), jnp.int32)
  lane_iota = lax.iota(jnp.int32, L)
  half_n = jnp.full((L,), N // 2, jnp.int32)

  def clear2(ha_ref, hb_ref, nv):
    def body(j, _):
      ha_ref[pl.ds(j * L, L)] = zero_i
      hb_ref[pl.ds(j * L, L)] = zero_i
      return 0
    lax.fori_loop(0, nv, body, 0, unroll=8)

  clear2(h1a, h1b, R1 // L)
  clear2(h2a, h2b, 2 * R23 // L)
  clear2(h3a, h3b, 2 * R23 // L)

  def keygen(i, carry):
    maxv, s1v, sev = carry
    sa = pl.ds(i * L, L)
    sb = pl.ds((NH + i) * L, L)
    ta = time_c[sa]
    tb = time_c[sb]
    ra = risk_c[sa]
    rb = risk_c[sb]
    ea = ev_c[sa]
    eb = ev_c[sb]
    ka = 0x3F7FFFFF - plsc.bitcast(ta, jnp.int32)
    kb = 0x3F7FFFFF - plsc.bitcast(tb, jnp.int32)
    key_a[sa] = ka
    key_a[sb] = kb
    plsc.addupdate_scatter(h1a, [jnp.bitwise_and(ka, R1 - 1)], one_i)
    plsc.addupdate_scatter(h1b, [jnp.bitwise_and(kb, R1 - 1)], one_i)
    return (jnp.maximum(jnp.maximum(maxv, ra), rb),
            s1v + ra * ea + rb * eb, sev + ea + eb)

  maxv, s1v, sev = lax.fori_loop(
      0, NH, keygen, (jnp.full((L,), -jnp.inf, jnp.float32), zero_f, zero_f),
      unroll=4)
  gamma = jnp.max(maxv)

  def hist_scan2(ha_ref, hb_ref, nv):
    def body(j, carry):
      sl = pl.ds(j * L, L)
      a = ha_ref[sl]
      b = hb_ref[sl]
      h = a + b
      inc = plsc.cumsum(h)
      base = inc - h + carry
      offa[sl] = base
      offb[sl] = base + a
      return carry + jnp.sum(h)
    lax.fori_loop(0, nv, body, jnp.int32(0), unroll=4)

  def hist_scan4(ha_ref, hb_ref, nv):
    def body(j, carry):
      sl0 = pl.ds(j * L, L)
      sl1 = pl.ds(R23 + j * L, L)
      a = ha_ref[sl0] + hb_ref[sl0]
      b = ha_ref[sl1] + hb_ref[sl1]
      h = a + b
      inc = plsc.cumsum(h)
      base = inc - h + carry
      offa[sl0] = base
      offb[sl0] = base + a
      return carry + jnp.sum(h)
    lax.fori_loop(0, nv, body, jnp.int32(0), unroll=4)

  hist_scan2(h1a, h1b, R1 // L)

  def perm1(i, _):
    ka = key_a[pl.ds(i * L, L)]
    kb = key_a[pl.ds((NH + i) * L, L)]
    da = jnp.bitwise_and(ka, R1 - 1)
    db = jnp.bitwise_and(kb, R1 - 1)
    occa, lasta = plsc.scan_count(da)
    occb, lastb = plsc.scan_count(db)
    basea = plsc.load_gather(offa, [da])
    baseb = plsc.load_gather(offb, [db])
    posa = basea + occa - 1
    posb = baseb + occb - 1
    packa = jnp.bitwise_or(
        jnp.left_shift(jnp.right_shift(ka, R1_BITS), IDX_BITS),
        i * L + lane_iota)
    packb = jnp.bitwise_or(
        jnp.left_shift(jnp.right_shift(kb, R1_BITS), IDX_BITS),
        (NH + i) * L + lane_iota)
    plsc.store_scatter(work_b, [posa], packa)
    plsc.store_scatter(work_b, [posb], packb)
    plsc.store_scatter(offa, [da], basea + occa, mask=lasta)
    plsc.store_scatter(offb, [db], baseb + occb, mask=lastb)
    d2a = jnp.bitwise_and(jnp.right_shift(ka, R1_BITS), R23 - 1)
    d2b = jnp.bitwise_and(jnp.right_shift(kb, R1_BITS), R23 - 1)
    ba = jnp.bitwise_or(jnp.left_shift(jnp.right_shift(posa, 13), R23_BITS),
                        d2a)
    bb = jnp.bitwise_or(jnp.left_shift(jnp.right_shift(posb, 13), R23_BITS),
                        d2b)
    plsc.addupdate_scatter(h2a, [ba], one_i)
    plsc.addupdate_scatter(h2b, [bb], one_i)
    return 0
  lax.fori_loop(0, NH, perm1, 0, unroll=4)

  hist_scan4(h2a, h2b, R23 // L)

  def perm2(i, _):
    pa = work_b[pl.ds(i * L, L)]
    pb = work_b[pl.ds((NH + i) * L, L)]
    da = jnp.bitwise_and(jnp.right_shift(pa, IDX_BITS), R23 - 1)
    db = jnp.bitwise_and(jnp.right_shift(pb, IDX_BITS), R23 - 1)
    occa, lasta = plsc.scan_count(da)
    occb, lastb = plsc.scan_count(db)
    basea = plsc.load_gather(offa, [da])
    baseb = plsc.load_gather(offb, [db])
    posa = basea + occa - 1
    posb = baseb + occb - 1
    plsc.store_scatter(key_a, [posa], pa)
    plsc.store_scatter(key_a, [posb], pb)
    plsc.store_scatter(offa, [da], basea + occa, mask=lasta)
    plsc.store_scatter(offb, [db], baseb + occb, mask=lastb)
    d3a = jnp.bitwise_and(jnp.right_shift(pa, IDX_BITS + R23_BITS), R23 - 1)
    d3b = jnp.bitwise_and(jnp.right_shift(pb, IDX_BITS + R23_BITS), R23 - 1)
    ba = jnp.bitwise_or(jnp.left_shift(jnp.right_shift(posa, 13), R23_BITS),
                        d3a)
    bb = jnp.bitwise_or(jnp.left_shift(jnp.right_shift(posb, 13), R23_BITS),
                        d3b)
    plsc.addupdate_scatter(h3a, [ba], one_i)
    plsc.addupdate_scatter(h3b, [bb], one_i)
    return 0
  lax.fori_loop(0, NH, perm2, 0, unroll=4)

  hist_scan4(h3a, h3b, R23 // L)

  def perm3(i, _):
    pa = key_a[pl.ds(i * L, L)]
    pb = key_a[pl.ds((NH + i) * L, L)]
    da = jnp.bitwise_and(jnp.right_shift(pa, IDX_BITS + R23_BITS), R23 - 1)
    db = jnp.bitwise_and(jnp.right_shift(pb, IDX_BITS + R23_BITS), R23 - 1)
    occa, lasta = plsc.scan_count(da)
    occb, lastb = plsc.scan_count(db)
    basea = plsc.load_gather(offa, [da])
    baseb = plsc.load_gather(offb, [db])
    posa = basea + occa - 1
    posb = baseb + occb - 1
    plsc.store_scatter(work_b, [posa], pa)
    plsc.store_scatter(work_b, [posb], pb)
    plsc.store_scatter(offa, [da], basea + occa, mask=lasta)
    plsc.store_scatter(offb, [db], baseb + occb, mask=lastb)
    return 0
  lax.fori_loop(0, NH, perm3, 0, unroll=4)

  def cox_body(i, carry):
    c0, acc2 = carry
    iv = jnp.bitwise_and(work_b[pl.ds(i * L, L)], IDX_MASK)
    r = plsc.load_gather(risk_c, [iv])
    e = plsc.load_gather(ev_c, [iv])
    x = jnp.exp(r - gamma)
    cs_raw = plsc.cumsum(x)
    lg = _log_poly(cs_raw + c0 + 1e-10)
    return (c0 + jnp.sum(x), acc2 + e * lg)

  c0, acc2 = lax.fori_loop(0, NV, cox_body, (jnp.float32(0.0), zero_f),
                           unroll=8)

  pvec[...] = s1v - acc2 - gamma * sev
  pltpu.sync_copy(pvec, out_hbm.at[wid])


@jax.jit
def _cox_loss(time_t, risk_t, ev_t):
  mesh = plsc.VectorSubcoreMesh(core_axis_name="c", subcore_axis_name="s")
  f = pl.kernel(
      _sc_body,
      out_type=jax.ShapeDtypeStruct((M, L), jnp.float32),
      mesh=mesh,
      scratch_types=[
          pltpu.VMEM((N,), jnp.float32),
          pltpu.VMEM((N,), jnp.float32),
          pltpu.VMEM((N,), jnp.float32),
          pltpu.VMEM((N,), jnp.int32),
          pltpu.VMEM((N,), jnp.int32),
          pltpu.VMEM((R1,), jnp.int32),
          pltpu.VMEM((R1,), jnp.int32),
          pltpu.VMEM((2 * R23,), jnp.int32),
          pltpu.VMEM((2 * R23,), jnp.int32),
          pltpu.VMEM((2 * R23,), jnp.int32),
          pltpu.VMEM((2 * R23,), jnp.int32),
          pltpu.VMEM((R1,), jnp.int32),
          pltpu.VMEM((R1,), jnp.int32),
          pltpu.VMEM((L,), jnp.float32),
      ],
      compiler_params=pltpu.CompilerParams(needs_layout_passes=False),
  )
  out = f(time_t, risk_t, ev_t)
  return -(jnp.sum(out) / (N * M))


def kernel(risk_pred, time, event):
  return _cox_loss(time.T, risk_pred.T, event.T)


# speedup vs baseline: 1.0083x; 1.0083x over previous
import jax
import jax.numpy as jnp
from jax import lax
from jax.experimental import pallas as pl
from jax.experimental.pallas import tpu as pltpu
from jax.experimental.pallas import tpu_sc as plsc

N = 16384
M = 32
L = 16
NV = N // L
NH = NV // 2
R1_BITS = 12
R1 = 1 << R1_BITS
R23_BITS = 9
R23 = 1 << R23_BITS
IDX_BITS = 14
IDX_MASK = (1 << IDX_BITS) - 1

_LN2 = 0.6931471805599453
_SQRT2 = 1.4142135623730951


def _log_poly(x):
  bits = plsc.bitcast(x, jnp.int32)
  e = jnp.right_shift(bits, 23) - 127
  m = plsc.bitcast(
      jnp.bitwise_or(jnp.bitwise_and(bits, 0x7FFFFF), 0x3F800000),
      jnp.float32)
  big = m > _SQRT2
  m = jnp.where(big, m * 0.5, m)
  e = e + jnp.where(big, 1, 0)
  s = (m - 1.0) / (m + 1.0)
  s2 = s * s
  p = 1.0 + s2 * (1.0 / 3.0 + s2 * (0.2 + s2 * (1.0 / 7.0 + s2 / 9.0)))
  return e.astype(jnp.float32) * _LN2 + 2.0 * s * p


def _sc_body(time_hbm, risk_hbm, ev_hbm, out_hbm,
             time_c, risk_c, ev_c, key_a, work_b,
             h1a, h1b, h2a, h2b, h3a, h3b, offa, offb, pvec):
  wid = lax.axis_index("s") * 2 + lax.axis_index("c")

  pltpu.sync_copy(time_hbm.at[wid], time_c)
  pltpu.sync_copy(risk_hbm.at[wid], risk_c)
  pltpu.sync_copy(ev_hbm.at[wid], ev_c)

  zero_i = jnp.zeros((L,), jnp.int32)
  zero_f = jnp.zeros((L,), jnp.float32)
  one_i = jnp.ones((L,), jnp.int32)
  lane_iota = lax.iota(jnp.int32, L)
  half_n = jnp.full((L,), N // 2, jnp.int32)

  def clear2(ha_ref, hb_ref, nv):
    def body(j, _):
      ha_ref[pl.ds(j * L, L)] = zero_i
      hb_ref[pl.ds(j * L, L)] = zero_i
      return 0
    lax.fori_loop(0, nv, body, 0, unroll=8)

  clear2(h1a, h1b, R1 // L)
  clear2(h2a, h2b, 2 * R23 // L)
  clear2(h3a, h3b, 2 * R23 // L)

  def keygen(i, carry):
    maxv, s1v, sev = carry
    sa = pl.ds(i * L, L)
    sb = pl.ds((NH + i) * L, L)
    ta = time_c[sa]
    tb = time_c[sb]
    ra = risk_c[sa]
    rb = risk_c[sb]
    ea = ev_c[sa]
    eb = ev_c[sb]
    ka = 0x3F7FFFFF - plsc.bitcast(ta, jnp.int32)
    kb = 0x3F7FFFFF - plsc.bitcast(tb, jnp.int32)
    key_a[sa] = ka
    key_a[sb] = kb
    plsc.addupdate_scatter(h1a, [jnp.bitwise_and(ka, R1 - 1)], one_i)
    plsc.addupdate_scatter(h1b, [jnp.bitwise_and(kb, R1 - 1)], one_i)
    return (jnp.maximum(jnp.maximum(maxv, ra), rb),
            s1v + ra * ea + rb * eb, sev + ea + eb)

  maxv, s1v, sev = lax.fori_loop(
      0, NH, keygen, (jnp.full((L,), -jnp.inf, jnp.float32), zero_f, zero_f),
      unroll=4)
  gamma = jnp.max(maxv)

  def hist_scan2(ha_ref, hb_ref, nv):
    def body(j, carry):
      sl = pl.ds(j * L, L)
      a = ha_ref[sl]
      b = hb_ref[sl]
      h = a + b
      inc = plsc.cumsum(h)
      base = inc - h + carry
      offa[sl] = base
      offb[sl] = base + a
      return carry + jnp.sum(h)
    lax.fori_loop(0, nv, body, jnp.int32(0), unroll=4)

  def hist_scan4(ha_ref, hb_ref, nv):
    def body(j, carry):
      sl0 = pl.ds(j * L, L)
      sl1 = pl.ds(R23 + j * L, L)
      a = ha_ref[sl0] + hb_ref[sl0]
      b = ha_ref[sl1] + hb_ref[sl1]
      h = a + b
      inc = plsc.cumsum(h)
      base = inc - h + carry
      offa[sl0] = base
      offb[sl0] = base + a
      return carry + jnp.sum(h)
    lax.fori_loop(0, nv, body, jnp.int32(0), unroll=4)

  hist_scan2(h1a, h1b, R1 // L)

  def perm1(i, _):
    ka = key_a[pl.ds(i * L, L)]
    kb = key_a[pl.ds((NH + i) * L, L)]
    da = jnp.bitwise_and(ka, R1 - 1)
    db = jnp.bitwise_and(kb, R1 - 1)
    occa, lasta = plsc.scan_count(da)
    occb, lastb = plsc.scan_count(db)
    basea = plsc.load_gather(offa, [da])
    baseb = plsc.load_gather(offb, [db])
    posa = basea + occa - 1
    posb = baseb + occb - 1
    packa = jnp.bitwise_or(
        jnp.left_shift(jnp.right_shift(ka, R1_BITS), IDX_BITS),
        i * L + lane_iota)
    packb = jnp.bitwise_or(
        jnp.left_shift(jnp.right_shift(kb, R1_BITS), IDX_BITS),
        (NH + i) * L + lane_iota)
    plsc.store_scatter(work_b, [posa], packa)
    plsc.store_scatter(work_b, [posb], packb)
    plsc.store_scatter(offa, [da], basea + occa, mask=lasta)
    plsc.store_scatter(offb, [db], baseb + occb, mask=lastb)
    d2a = jnp.bitwise_and(jnp.right_shift(ka, R1_BITS), R23 - 1)
    d2b = jnp.bitwise_and(jnp.right_shift(kb, R1_BITS), R23 - 1)
    ba = jnp.bitwise_or(jnp.left_shift(jnp.right_shift(posa, 13), R23_BITS),
                        d2a)
    bb = jnp.bitwise_or(jnp.left_shift(jnp.right_shift(posb, 13), R23_BITS),
                        d2b)
    plsc.addupdate_scatter(h2a, [ba], one_i)
    plsc.addupdate_scatter(h2b, [bb], one_i)
    return 0
  lax.fori_loop(0, NH, perm1, 0, unroll=2)

  hist_scan4(h2a, h2b, R23 // L)

  def perm2(i, _):
    pa = work_b[pl.ds(i * L, L)]
    pb = work_b[pl.ds((NH + i) * L, L)]
    da = jnp.bitwise_and(jnp.right_shift(pa, IDX_BITS), R23 - 1)
    db = jnp.bitwise_and(jnp.right_shift(pb, IDX_BITS), R23 - 1)
    occa, lasta = plsc.scan_count(da)
    occb, lastb = plsc.scan_count(db)
    basea = plsc.load_gather(offa, [da])
    baseb = plsc.load_gather(offb, [db])
    posa = basea + occa - 1
    posb = baseb + occb - 1
    plsc.store_scatter(key_a, [posa], pa)
    plsc.store_scatter(key_a, [posb], pb)
    plsc.store_scatter(offa, [da], basea + occa, mask=lasta)
    plsc.store_scatter(offb, [db], baseb + occb, mask=lastb)
    d3a = jnp.bitwise_and(jnp.right_shift(pa, IDX_BITS + R23_BITS), R23 - 1)
    d3b = jnp.bitwise_and(jnp.right_shift(pb, IDX_BITS + R23_BITS), R23 - 1)
    ba = jnp.bitwise_or(jnp.left_shift(jnp.right_shift(posa, 13), R23_BITS),
                        d3a)
    bb = jnp.bitwise_or(jnp.left_shift(jnp.right_shift(posb, 13), R23_BITS),
                        d3b)
    plsc.addupdate_scatter(h3a, [ba], one_i)
    plsc.addupdate_scatter(h3b, [bb], one_i)
    return 0
  lax.fori_loop(0, NH, perm2, 0, unroll=2)

  hist_scan4(h3a, h3b, R23 // L)

  def perm3(i, _):
    pa = key_a[pl.ds(i * L, L)]
    pb = key_a[pl.ds((NH + i) * L, L)]
    da = jnp.bitwise_and(jnp.right_shift(pa, IDX_BITS + R23_BITS), R23 - 1)
    db = jnp.bitwise_and(jnp.right_shift(pb, IDX_BITS + R23_BITS), R23 - 1)
    occa, lasta = plsc.scan_count(da)
    occb, lastb = plsc.scan_count(db)
    basea = plsc.load_gather(offa, [da])
    baseb = plsc.load_gather(offb, [db])
    posa = basea + occa - 1
    posb = baseb + occb - 1
    plsc.store_scatter(work_b, [posa], pa)
    plsc.store_scatter(work_b, [posb], pb)
    plsc.store_scatter(offa, [da], basea + occa, mask=lasta)
    plsc.store_scatter(offb, [db], baseb + occb, mask=lastb)
    return 0
  lax.fori_loop(0, NH, perm3, 0, unroll=2)

  def cox_body(i, carry):
    c0, acc2 = carry
    iv = jnp.bitwise_and(work_b[pl.ds(i * L, L)], IDX_MASK)
    r = plsc.load_gather(risk_c, [iv])
    e = plsc.load_gather(ev_c, [iv])
    x = jnp.exp(r - gamma)
    cs_raw = plsc.cumsum(x)
    lg = _log_poly(cs_raw + c0 + 1e-10)
    return (c0 + jnp.sum(x), acc2 + e * lg)

  c0, acc2 = lax.fori_loop(0, NV, cox_body, (jnp.float32(0.0), zero_f),
                           unroll=4)

  pvec[...] = s1v - acc2 - gamma * sev
  pltpu.sync_copy(pvec, out_hbm.at[wid])


@jax.jit
def _cox_loss(time_t, risk_t, ev_t):
  mesh = plsc.VectorSubcoreMesh(core_axis_name="c", subcore_axis_name="s")
  f = pl.kernel(
      _sc_body,
      out_type=jax.ShapeDtypeStruct((M, L), jnp.float32),
      mesh=mesh,
      scratch_types=[
          pltpu.VMEM((N,), jnp.float32),
          pltpu.VMEM((N,), jnp.float32),
          pltpu.VMEM((N,), jnp.float32),
          pltpu.VMEM((N,), jnp.int32),
          pltpu.VMEM((N,), jnp.int32),
          pltpu.VMEM((R1,), jnp.int32),
          pltpu.VMEM((R1,), jnp.int32),
          pltpu.VMEM((2 * R23,), jnp.int32),
          pltpu.VMEM((2 * R23,), jnp.int32),
          pltpu.VMEM((2 * R23,), jnp.int32),
          pltpu.VMEM((2 * R23,), jnp.int32),
          pltpu.VMEM((R1,), jnp.int32),
          pltpu.VMEM((R1,), jnp.int32),
          pltpu.VMEM((L,), jnp.float32),
      ],
      compiler_params=pltpu.CompilerParams(needs_layout_passes=False),
  )
  out = f(time_t, risk_t, ev_t)
  return -(jnp.sum(out) / (N * M))


def kernel(risk_pred, time, event):
  return _cox_loss(time.T, risk_pred.T, event.T)


# speedup vs baseline: 1.0088x; 1.0005x over previous
import jax
import jax.numpy as jnp
from jax import lax
from jax.experimental import pallas as pl
from jax.experimental.pallas import tpu as pltpu
from jax.experimental.pallas import tpu_sc as plsc

N = 16384
M = 32
L = 16
NV = N // L
NH = NV // 2
R1_BITS = 12
R1 = 1 << R1_BITS
R23_BITS = 9
R23 = 1 << R23_BITS
IDX_BITS = 14
IDX_MASK = (1 << IDX_BITS) - 1

_LN2 = 0.6931471805599453
_SQRT2 = 1.4142135623730951


def _log_poly(x):
  bits = plsc.bitcast(x, jnp.int32)
  e = jnp.right_shift(bits, 23) - 127
  m = plsc.bitcast(
      jnp.bitwise_or(jnp.bitwise_and(bits, 0x7FFFFF), 0x3F800000),
      jnp.float32)
  big = m > _SQRT2
  m = jnp.where(big, m * 0.5, m)
  e = e + jnp.where(big, 1, 0)
  s = (m - 1.0) / (m + 1.0)
  s2 = s * s
  p = 1.0 + s2 * (1.0 / 3.0 + s2 * (0.2 + s2 * (1.0 / 7.0 + s2 / 9.0)))
  return e.astype(jnp.float32) * _LN2 + 2.0 * s * p


def _sc_body(time_hbm, risk_hbm, ev_hbm, out_hbm,
             time_c, risk_c, ev_c, key_a, work_b,
             h1a, h1b, h2a, h2b, h3a, h3b, offa, offb, pvec):
  wid = lax.axis_index("s") * 2 + lax.axis_index("c")

  pltpu.sync_copy(time_hbm.at[wid], time_c)
  pltpu.sync_copy(risk_hbm.at[wid], risk_c)
  pltpu.sync_copy(ev_hbm.at[wid], ev_c)

  zero_i = jnp.zeros((L,), jnp.int32)
  zero_f = jnp.zeros((L,), jnp.float32)
  one_i = jnp.ones((L,), jnp.int32)
  lane_iota = lax.iota(jnp.int32, L)
  half_n = jnp.full((L,), N // 2, jnp.int32)

  def clear2(ha_ref, hb_ref, nv):
    def body(j, _):
      ha_ref[pl.ds(j * L, L)] = zero_i
      hb_ref[pl.ds(j * L, L)] = zero_i
      return 0
    lax.fori_loop(0, nv, body, 0, unroll=8)

  clear2(h1a, h1b, R1 // L)
  clear2(h2a, h2b, 2 * R23 // L)
  clear2(h3a, h3b, 2 * R23 // L)

  def keygen(i, carry):
    maxv, s1v, sev = carry
    sa = pl.ds(i * L, L)
    sb = pl.ds((NH + i) * L, L)
    ta = time_c[sa]
    tb = time_c[sb]
    ra = risk_c[sa]
    rb = risk_c[sb]
    ea = ev_c[sa]
    eb = ev_c[sb]
    ka = 0x3F7FFFFF - plsc.bitcast(ta, jnp.int32)
    kb = 0x3F7FFFFF - plsc.bitcast(tb, jnp.int32)
    key_a[sa] = ka
    key_a[sb] = kb
    plsc.addupdate_scatter(h1a, [jnp.bitwise_and(ka, R1 - 1)], one_i)
    plsc.addupdate_scatter(h1b, [jnp.bitwise_and(kb, R1 - 1)], one_i)
    return (jnp.maximum(jnp.maximum(maxv, ra), rb),
            s1v + ra * ea + rb * eb, sev + ea + eb)

  maxv, s1v, sev = lax.fori_loop(
      0, NH, keygen, (jnp.full((L,), -jnp.inf, jnp.float32), zero_f, zero_f),
      unroll=4)
  gamma = jnp.max(maxv)

  def hist_scan2(ha_ref, hb_ref, nv):
    def body(j, carry):
      sl = pl.ds(j * L, L)
      a = ha_ref[sl]
      b = hb_ref[sl]
      h = a + b
      inc = plsc.cumsum(h)
      base = inc - h + carry
      offa[sl] = base
      offb[sl] = base + a
      return carry + jnp.sum(h)
    lax.fori_loop(0, nv, body, jnp.int32(0), unroll=4)

  def hist_scan4(ha_ref, hb_ref, nv):
    def body(j, carry):
      sl0 = pl.ds(j * L, L)
      sl1 = pl.ds(R23 + j * L, L)
      a = ha_ref[sl0] + hb_ref[sl0]
      b = ha_ref[sl1] + hb_ref[sl1]
      h = a + b
      inc = plsc.cumsum(h)
      base = inc - h + carry
      offa[sl0] = base
      offb[sl0] = base + a
      return carry + jnp.sum(h)
    lax.fori_loop(0, nv, body, jnp.int32(0), unroll=4)

  hist_scan2(h1a, h1b, R1 // L)

  def perm1(i, _):
    ka = key_a[pl.ds(i * L, L)]
    kb = key_a[pl.ds((NH + i) * L, L)]
    da = jnp.bitwise_and(ka, R1 - 1)
    db = jnp.bitwise_and(kb, R1 - 1)
    occa, lasta = plsc.scan_count(da)
    occb, lastb = plsc.scan_count(db)
    basea = plsc.load_gather(offa, [da])
    baseb = plsc.load_gather(offb, [db])
    posa = basea + occa - 1
    posb = baseb + occb - 1
    packa = jnp.bitwise_or(
        jnp.left_shift(jnp.right_shift(ka, R1_BITS), IDX_BITS),
        i * L + lane_iota)
    packb = jnp.bitwise_or(
        jnp.left_shift(jnp.right_shift(kb, R1_BITS), IDX_BITS),
        (NH + i) * L + lane_iota)
    plsc.store_scatter(work_b, [posa], packa)
    plsc.store_scatter(work_b, [posb], packb)
    plsc.store_scatter(offa, [da], basea + occa, mask=lasta)
    plsc.store_scatter(offb, [db], baseb + occb, mask=lastb)
    d2a = jnp.bitwise_and(jnp.right_shift(ka, R1_BITS), R23 - 1)
    d2b = jnp.bitwise_and(jnp.right_shift(kb, R1_BITS), R23 - 1)
    ba = jnp.bitwise_or(jnp.left_shift(jnp.right_shift(posa, 13), R23_BITS),
                        d2a)
    bb = jnp.bitwise_or(jnp.left_shift(jnp.right_shift(posb, 13), R23_BITS),
                        d2b)
    plsc.addupdate_scatter(h2a, [ba], one_i)
    plsc.addupdate_scatter(h2b, [bb], one_i)
    return 0
  lax.fori_loop(0, NH, perm1, 0, unroll=2)

  hist_scan4(h2a, h2b, R23 // L)

  def perm2(i, _):
    pa = work_b[pl.ds(i * L, L)]
    pb = work_b[pl.ds((NH + i) * L, L)]
    da = jnp.bitwise_and(jnp.right_shift(pa, IDX_BITS), R23 - 1)
    db = jnp.bitwise_and(jnp.right_shift(pb, IDX_BITS), R23 - 1)
    occa, lasta = plsc.scan_count(da)
    occb, lastb = plsc.scan_count(db)
    basea = plsc.load_gather(offa, [da])
    baseb = plsc.load_gather(offb, [db])
    posa = basea + occa - 1
    posb = baseb + occb - 1
    plsc.store_scatter(key_a, [posa], pa)
    plsc.store_scatter(key_a, [posb], pb)
    plsc.store_scatter(offa, [da], basea + occa, mask=lasta)
    plsc.store_scatter(offb, [db], baseb + occb, mask=lastb)
    d3a = jnp.bitwise_and(jnp.right_shift(pa, IDX_BITS + R23_BITS), R23 - 1)
    d3b = jnp.bitwise_and(jnp.right_shift(pb, IDX_BITS + R23_BITS), R23 - 1)
    ba = jnp.bitwise_or(jnp.left_shift(jnp.right_shift(posa, 13), R23_BITS),
                        d3a)
    bb = jnp.bitwise_or(jnp.left_shift(jnp.right_shift(posb, 13), R23_BITS),
                        d3b)
    plsc.addupdate_scatter(h3a, [ba], one_i)
    plsc.addupdate_scatter(h3b, [bb], one_i)
    return 0
  lax.fori_loop(0, NH, perm2, 0, unroll=2)

  hist_scan4(h3a, h3b, R23 // L)

  def perm3(i, _):
    pa = key_a[pl.ds(i * L, L)]
    pb = key_a[pl.ds((NH + i) * L, L)]
    da = jnp.bitwise_and(jnp.right_shift(pa, IDX_BITS + R23_BITS), R23 - 1)
    db = jnp.bitwise_and(jnp.right_shift(pb, IDX_BITS + R23_BITS), R23 - 1)
    occa, lasta = plsc.scan_count(da)
    occb, lastb = plsc.scan_count(db)
    basea = plsc.load_gather(offa, [da])
    baseb = plsc.load_gather(offb, [db])
    posa = basea + occa - 1
    posb = baseb + occb - 1
    plsc.store_scatter(work_b, [posa], pa)
    plsc.store_scatter(work_b, [posb], pb)
    plsc.store_scatter(offa, [da], basea + occa, mask=lasta)
    plsc.store_scatter(offb, [db], baseb + occb, mask=lastb)
    return 0
  lax.fori_loop(0, NH, perm3, 0, unroll=2)

  def cox_body(i, carry):
    c0, acc2 = carry
    iv = jnp.bitwise_and(work_b[pl.ds(i * L, L)], IDX_MASK)
    r = plsc.load_gather(risk_c, [iv])
    e = plsc.load_gather(ev_c, [iv])
    x = jnp.exp(r - gamma)
    cs_raw = plsc.cumsum(x)
    lg = _log_poly(cs_raw + c0 + 1e-10)
    return (c0 + jnp.sum(x), acc2 + e * lg)

  c0, acc2 = lax.fori_loop(0, NV, cox_body, (jnp.float32(0.0), zero_f),
                           unroll=2)

  pvec[...] = s1v - acc2 - gamma * sev
  pltpu.sync_copy(pvec, out_hbm.at[wid])


@jax.jit
def _cox_loss(time_t, risk_t, ev_t):
  mesh = plsc.VectorSubcoreMesh(core_axis_name="c", subcore_axis_name="s")
  f = pl.kernel(
      _sc_body,
      out_type=jax.ShapeDtypeStruct((M, L), jnp.float32),
      mesh=mesh,
      scratch_types=[
          pltpu.VMEM((N,), jnp.float32),
          pltpu.VMEM((N,), jnp.float32),
          pltpu.VMEM((N,), jnp.float32),
          pltpu.VMEM((N,), jnp.int32),
          pltpu.VMEM((N,), jnp.int32),
          pltpu.VMEM((R1,), jnp.int32),
          pltpu.VMEM((R1,), jnp.int32),
          pltpu.VMEM((2 * R23,), jnp.int32),
          pltpu.VMEM((2 * R23,), jnp.int32),
          pltpu.VMEM((2 * R23,), jnp.int32),
          pltpu.VMEM((2 * R23,), jnp.int32),
          pltpu.VMEM((R1,), jnp.int32),
          pltpu.VMEM((R1,), jnp.int32),
          pltpu.VMEM((L,), jnp.float32),
      ],
      compiler_params=pltpu.CompilerParams(needs_layout_passes=False),
  )
  out = f(time_t, risk_t, ev_t)
  return -(jnp.sum(out) / (N * M))


def kernel(risk_pred, time, event):
  return _cox_loss(time.T, risk_pred.T, event.T)
